# baseline (speedup 1.0000x reference)
import functools
import jax
import jax.numpy as jnp
from jax import lax
from jax.experimental import pallas as pl
from jax.experimental.pallas import tpu as pltpu
from jax.experimental.pallas import tpu_sc as plsc
import numpy as np

_N = 10000
_NPAD = 10240
_NW = 16
_PP = _NPAD // _NW
_NBLK = _PP // 16
_CAND = 4 * _NPAD
_LEAF = _CAND // 16
_L2 = _LEAF // 16
_SCORE_THRESH = 0.05
_NMS_THRESH = 0.5
_DETS = 100
_IMG = 512.0
_CLIP = float(np.log(1000.0 / 16.0))


def _lane0_store(ref, i, val, lane):
    idx = jnp.full((16,), i, jnp.int32)
    if getattr(val, "shape", ()) != (16,):
        val = jnp.full((16,), val, ref.dtype)
    plsc.store_scatter(ref, [idx], val, mask=lane == 0)


def _sc_body(cl_h, br_h, pr_h,
             ox1_h, oy1_h, ox2_h, oy2_h, osc_h, olab_h,
             cl_v, br_v, pr_v, sL, bL, mxv, l1L,
             Ss, SL1, Sbox, Smax,
             sc_v, L1, L2r, L3r, smax_v, bbuf, sem,
             kx1r, ky1r, kx2r, ky2r, karr,
             ox1, oy1, ox2, oy2, osc, olab):
    wid = lax.axis_index("s")
    lane = lax.broadcasted_iota(jnp.int32, (16,), 0)

    base = wid * _PP
    pltpu.sync_copy(cl_h.at[pl.ds(base * 5, _PP * 5)], cl_v)
    pltpu.sync_copy(br_h.at[pl.ds(base * 20, _PP * 20)], br_v)
    pltpu.sync_copy(pr_h.at[pl.ds(base * 4, _PP * 4)], pr_v)

    def pa(g, mx):
        sl = pl.ds(g * 16, 16)
        rows = g * 16 + lane

        def gcol(ref, ncol, j):
            return plsc.load_gather(ref, [rows * ncol + j])

        p0 = gcol(pr_v, 4, 0)
        p1 = gcol(pr_v, 4, 1)
        p2 = gcol(pr_v, 4, 2)
        p3 = gcol(pr_v, 4, 3)
        w = p2 - p0
        h = p3 - p1
        cx = p0 + 0.5 * w
        cy = p1 + 0.5 * h
        l0 = gcol(cl_v, 5, 0)
        l1 = gcol(cl_v, 5, 1)
        l2 = gcol(cl_v, 5, 2)
        l3 = gcol(cl_v, 5, 3)
        l4 = gcol(cl_v, 5, 4)
        lm = jnp.maximum(jnp.maximum(jnp.maximum(l0, l1), jnp.maximum(l2, l3)), l4)
        e0 = jnp.exp(l0 - lm)
        e1 = jnp.exp(l1 - lm)
        e2 = jnp.exp(l2 - lm)
        e3 = jnp.exp(l3 - lm)
        e4 = jnp.exp(l4 - lm)
        den = e0 + e1 + e2 + e3 + e4
        real = (base + g * 16 + lane) < _N
        es = [e1, e2, e3, e4]
        for c in range(1, 5):
            dx = gcol(br_v, 20, 4 * c + 0) / 10.0
            dy = gcol(br_v, 20, 4 * c + 1) / 10.0
            dw = jnp.minimum(gcol(br_v, 20, 4 * c + 2) / 5.0, _CLIP)
            dh = jnp.minimum(gcol(br_v, 20, 4 * c + 3) / 5.0, _CLIP)
            px = dx * w + cx
            py = dy * h + cy
            pw = jnp.exp(dw) * w
            ph = jnp.exp(dh) * h
            x1 = jnp.clip(px - 0.5 * pw, 0.0, _IMG)
            x2 = jnp.clip(px + 0.5 * pw, 0.0, _IMG)
            y1 = jnp.clip(py - 0.5 * ph, 0.0, _IMG)
            y2 = jnp.clip(py + 0.5 * ph, 0.0, _IMG)
            scr = es[c - 1] / den
            keep = ((scr > _SCORE_THRESH) & ((x2 - x1) >= 0.01)
                    & ((y2 - y1) >= 0.01) & real)
            ms = jnp.where(keep, scr, -1.0)
            _lane0_store(l1L, (c - 1) * _NBLK + g, jnp.max(ms), lane)
            sL[c - 1, sl] = ms
            bL[c - 1, pl.ds(g * 64, 16)] = x1
            bL[c - 1, pl.ds(g * 64 + 16, 16)] = y1
            bL[c - 1, pl.ds(g * 64 + 32, 16)] = x2
            bL[c - 1, pl.ds(g * 64 + 48, 16)] = y2
            mx = jnp.maximum(mx, jnp.maximum(jnp.maximum(x1, x2),
                                             jnp.maximum(y1, y2)))
        return mx

    mx = lax.fori_loop(0, _NBLK, pa, jnp.zeros((16,), jnp.float32))
    mxv[:] = mx

    for c in range(4):
        pltpu.sync_copy(sL.at[c], Ss.at[pl.ds(c * _NPAD + base, _PP)])
        pltpu.sync_copy(bL.at[c],
                        Sbox.at[pl.ds((c * (_NPAD // 16) + wid * _NBLK) * 64,
                                      _NBLK * 64)])
        pltpu.sync_copy(l1L.at[pl.ds(c * _NBLK, _NBLK)],
                        SL1.at[pl.ds(c * (_NPAD // 16) + wid * _NBLK, _NBLK)])
    pltpu.sync_copy(mxv, Smax.at[pl.ds(wid * 16, 16)])

    plsc.subcore_barrier()

    @pl.when(wid == 0)
    def _scan():
        pltpu.sync_copy(Ss, sc_v)
        pltpu.sync_copy(SL1, L1)
        pltpu.sync_copy(Smax, smax_v)
        m16 = smax_v[pl.ds(0, 16)]
        for i in range(1, _NW):
            m16 = jnp.maximum(m16, smax_v[pl.ds(i * 16, 16)])
        off_base = jnp.max(m16) + 1.0

        zf = jnp.zeros((16,), jnp.float32)
        zi = jnp.zeros((16,), jnp.int32)
        for j in range(8):
            sl = pl.ds(j * 16, 16)
            ox1[sl] = zf
            oy1[sl] = zf
            ox2[sl] = zf
            oy2[sl] = zf
            osc[sl] = zf
            olab[sl] = zi

        def bl2(chunk, _):
            idxb = chunk * 256 + lane * 16
            m = plsc.load_gather(L1, [idxb])
            for i in range(1, 16):
                m = jnp.maximum(m, plsc.load_gather(L1, [idxb + i]))
            L2r[pl.ds(chunk * 16, 16)] = m
            return 0

        lax.fori_loop(0, _L2 // 16, bl2, 0)

        for j in range(6):
            L2r[pl.ds(_L2 + j * 16, 16)] = jnp.full((16,), -2.0, jnp.float32)

        idxb = lane * 16
        m = plsc.load_gather(L2r, [idxb])
        for i in range(1, 16):
            m = jnp.maximum(m, plsc.load_gather(L2r, [idxb + i]))
        L3r[:] = m

        big = jnp.int32(9999)

        def cond(st):
            nk, stop = st
            return (nk < _DETS) & (stop == 0)

        def body(st):
            nk, stop = st
            v3 = L3r[:]
            g3 = jnp.max(v3)
            valid = g3 > 0.0
            l3 = plsc.all_reduce_ffs(v3 == g3)[0]
            v2 = L2r[pl.ds(l3 * 16, 16)]
            l2 = plsc.all_reduce_ffs(v2 == g3)[0]
            e2 = l3 * 16 + l2
            v1 = L1[pl.ds(e2 * 16, 16)]
            l1 = plsc.all_reduce_ffs(v1 == g3)[0]
            e1 = e2 * 16 + l1
            cp = pltpu.async_copy(Sbox.at[pl.ds(e1 * 64, 64)], bbuf, sem)
            v0 = sc_v[pl.ds(e1 * 16, 16)]
            l0 = plsc.all_reduce_ffs(v0 == g3)[0]
            k = e1 * 16 + l0

            @pl.when(valid)
            def _consume():
                v0n = jnp.where(lane == l0, -1.0, v0)
                _lane0_store(sc_v, k, jnp.float32(-1.0), lane)
                _lane0_store(L1, e1, jnp.max(v0n), lane)
                _lane0_store(L2r, e2, jnp.max(L1[pl.ds(e2 * 16, 16)]), lane)
                _lane0_store(L3r, l3, jnp.max(L2r[pl.ds(l3 * 16, 16)]), lane)

            cp.wait()

            def spl(q):
                return plsc.load_gather(bbuf, [jnp.full((16,), q, jnp.int32)])

            bo1 = spl(l0)
            bo2 = spl(l0 + 16)
            bo3 = spl(l0 + 32)
            bo4 = spl(l0 + 48)
            lab = k // _NPAD + 1
            off = lab.astype(jnp.float32) * off_base
            bx1 = bo1 + off
            by1 = bo2 + off
            bx2 = bo3 + off
            by2 = bo4 + off
            areac = (bx2 - bx1) * (by2 - by1)

            nblk = (nk + 15) // 16

            def iou_blk(j, hit):
                sl = pl.ds(j * 16, 16)
                kx1 = kx1r[sl]
                ky1 = ky1r[sl]
                kx2 = kx2r[sl]
                ky2 = ky2r[sl]
                kar = karr[sl]
                ltx = jnp.maximum(kx1, bx1)
                lty = jnp.maximum(ky1, by1)
                rbx = jnp.minimum(kx2, bx2)
                rby = jnp.minimum(ky2, by2)
                inter = (jnp.maximum(rbx - ltx, 0.0)
                         * jnp.maximum(rby - lty, 0.0))
                iou = inter / (kar + areac - inter + 1e-9)
                ok = (j * 16 + lane) < nk
                return hit | ((iou > _NMS_THRESH) & ok).astype(jnp.int32)

            hit = lax.fori_loop(0, nblk, iou_blk, jnp.zeros((16,), jnp.int32))
            rej = plsc.all_reduce_population_count(hit > 0)[0] > 0
            keep_it = valid & (~rej)

            @pl.when(keep_it)
            def _append():
                _lane0_store(kx1r, nk, bx1, lane)
                _lane0_store(ky1r, nk, by1, lane)
                _lane0_store(kx2r, nk, bx2, lane)
                _lane0_store(ky2r, nk, by2, lane)
                _lane0_store(karr, nk, areac, lane)
                _lane0_store(ox1, nk, bo1, lane)
                _lane0_store(oy1, nk, bo2, lane)
                _lane0_store(ox2, nk, bo3, lane)
                _lane0_store(oy2, nk, bo4, lane)
                _lane0_store(osc, nk, g3, lane)
                _lane0_store(olab, nk, lab, lane)

            nk2 = nk + keep_it.astype(jnp.int32)
            stop2 = jnp.where(valid, 0, 1).astype(jnp.int32)
            return nk2, stop2

        lax.while_loop(cond, body, (jnp.int32(0), jnp.int32(0)))

        pltpu.sync_copy(ox1, ox1_h)
        pltpu.sync_copy(oy1, oy1_h)
        pltpu.sync_copy(ox2, ox2_h)
        pltpu.sync_copy(oy2, oy2_h)
        pltpu.sync_copy(osc, osc_h)
        pltpu.sync_copy(olab, olab_h)


@jax.jit
def kernel(class_logits, box_regression, proposals):
    padn = _NPAD - _N
    clT = jnp.pad(class_logits, ((0, padn), (0, 0))).reshape(-1)
    brT = jnp.pad(box_regression, ((0, padn), (0, 0))).reshape(-1)
    prT = jnp.pad(proposals, ((0, padn), (0, 0))).reshape(-1)

    mesh = plsc.VectorSubcoreMesh(core_axis_name="c", subcore_axis_name="s",
                                  num_cores=1, num_subcores=_NW)
    f32 = jnp.float32
    run = pl.kernel(
        _sc_body,
        out_type=(
            jax.ShapeDtypeStruct((128,), f32),
            jax.ShapeDtypeStruct((128,), f32),
            jax.ShapeDtypeStruct((128,), f32),
            jax.ShapeDtypeStruct((128,), f32),
            jax.ShapeDtypeStruct((128,), f32),
            jax.ShapeDtypeStruct((128,), jnp.int32),
        ),
        mesh=mesh,
        compiler_params=pltpu.CompilerParams(needs_layout_passes=False),
        scratch_types=[
            pltpu.VMEM((_PP * 5,), f32),
            pltpu.VMEM((_PP * 20,), f32),
            pltpu.VMEM((_PP * 4,), f32),
            pltpu.VMEM((4, _PP), f32),
            pltpu.VMEM((4, _NBLK * 64), f32),
            pltpu.VMEM((16,), f32),
            pltpu.VMEM((4 * _NBLK,), f32),
            pltpu.VMEM_SHARED((_CAND,), f32),
            pltpu.VMEM_SHARED((_LEAF,), f32),
            pltpu.VMEM_SHARED((_LEAF * 64,), f32),
            pltpu.VMEM_SHARED((_NW * 16,), f32),
            pltpu.VMEM((_CAND,), f32),
            pltpu.VMEM((_LEAF,), f32),
            pltpu.VMEM((256,), f32),
            pltpu.VMEM((16,), f32),
            pltpu.VMEM((_NW * 16,), f32),
            pltpu.VMEM((64,), f32),
            pltpu.SemaphoreType.DMA,
            pltpu.VMEM((112,), f32),
            pltpu.VMEM((112,), f32),
            pltpu.VMEM((112,), f32),
            pltpu.VMEM((112,), f32),
            pltpu.VMEM((112,), f32),
            pltpu.VMEM((128,), f32),
            pltpu.VMEM((128,), f32),
            pltpu.VMEM((128,), f32),
            pltpu.VMEM((128,), f32),
            pltpu.VMEM((128,), f32),
            pltpu.VMEM((128,), jnp.int32),
        ],
    )
    x1, y1, x2, y2, sc, lab = run(clT, brT, prT)
    out_boxes = jnp.stack([x1, y1, x2, y2], axis=1)[:_DETS]
    return out_boxes, sc[:_DETS], lab[:_DETS]


# speedup vs baseline: 1.5544x; 1.5544x over previous
import functools
import jax
import jax.numpy as jnp
from jax import lax
from jax.experimental import pallas as pl
from jax.experimental.pallas import tpu as pltpu
from jax.experimental.pallas import tpu_sc as plsc
import numpy as np

_N = 10000
_NPAD = 10240
_NW = 16
_PP = _NPAD // _NW
_NBLK = _PP // 16
_CAND = 4 * _NPAD
_LEAF = _CAND // 16
_L2 = _LEAF // 16
_SCORE_THRESH = 0.05
_NMS_THRESH = 0.5
_DETS = 100
_IMG = 512.0
_CLIP = float(np.log(1000.0 / 16.0))


def _lane0_store(ref, i, val, lane):
    idx = jnp.full((16,), i, jnp.int32)
    if getattr(val, "shape", ()) != (16,):
        val = jnp.full((16,), val, ref.dtype)
    plsc.store_scatter(ref, [idx], val, mask=lane == 0)


def _sc_body(cl_h, br_h, pr_h,
             ox1_h, oy1_h, ox2_h, oy2_h, osc_h, olab_h,
             cl_v, br_v, pr_v, sL, bL, mxv, l1L,
             Ss, SL1, Sbox, Smax,
             sc_v, L1, L2r, L3r, smax_v, bbuf, sem,
             kx1r, ky1r, kx2r, ky2r, karr,
             ox1, oy1, ox2, oy2, osc, olab):
    wid = lax.axis_index("s")
    lane = lax.broadcasted_iota(jnp.int32, (16,), 0)

    base = wid * _PP
    pltpu.sync_copy(cl_h.at[:, pl.ds(base, _PP)], cl_v)
    pltpu.sync_copy(br_h.at[:, pl.ds(base, _PP)], br_v)
    pltpu.sync_copy(pr_h.at[:, pl.ds(base, _PP)], pr_v)

    def pa(g, mx):
        sl = pl.ds(g * 16, 16)
        p0 = pr_v[0, sl]
        p1 = pr_v[1, sl]
        p2 = pr_v[2, sl]
        p3 = pr_v[3, sl]
        w = p2 - p0
        h = p3 - p1
        cx = p0 + 0.5 * w
        cy = p1 + 0.5 * h
        l0 = cl_v[0, sl]
        l1 = cl_v[1, sl]
        l2 = cl_v[2, sl]
        l3 = cl_v[3, sl]
        l4 = cl_v[4, sl]
        lm = jnp.maximum(jnp.maximum(jnp.maximum(l0, l1), jnp.maximum(l2, l3)), l4)
        e0 = jnp.exp(l0 - lm)
        e1 = jnp.exp(l1 - lm)
        e2 = jnp.exp(l2 - lm)
        e3 = jnp.exp(l3 - lm)
        e4 = jnp.exp(l4 - lm)
        den = e0 + e1 + e2 + e3 + e4
        real = (base + g * 16 + lane) < _N
        es = [e1, e2, e3, e4]
        for c in range(1, 5):
            dx = br_v[4 * c + 0, sl] / 10.0
            dy = br_v[4 * c + 1, sl] / 10.0
            dw = jnp.minimum(br_v[4 * c + 2, sl] / 5.0, _CLIP)
            dh = jnp.minimum(br_v[4 * c + 3, sl] / 5.0, _CLIP)
            px = dx * w + cx
            py = dy * h + cy
            pw = jnp.exp(dw) * w
            ph = jnp.exp(dh) * h
            x1 = jnp.clip(px - 0.5 * pw, 0.0, _IMG)
            x2 = jnp.clip(px + 0.5 * pw, 0.0, _IMG)
            y1 = jnp.clip(py - 0.5 * ph, 0.0, _IMG)
            y2 = jnp.clip(py + 0.5 * ph, 0.0, _IMG)
            scr = es[c - 1] / den
            keep = ((scr > _SCORE_THRESH) & ((x2 - x1) >= 0.01)
                    & ((y2 - y1) >= 0.01) & real)
            ms = jnp.where(keep, scr, -1.0)
            _lane0_store(l1L, (c - 1) * _NBLK + g, jnp.max(ms), lane)
            sL[c - 1, sl] = ms
            bL[c - 1, pl.ds(g * 64, 16)] = x1
            bL[c - 1, pl.ds(g * 64 + 16, 16)] = y1
            bL[c - 1, pl.ds(g * 64 + 32, 16)] = x2
            bL[c - 1, pl.ds(g * 64 + 48, 16)] = y2
            mx = jnp.maximum(mx, jnp.maximum(jnp.maximum(x1, x2),
                                             jnp.maximum(y1, y2)))
        return mx

    mx = lax.fori_loop(0, _NBLK, pa, jnp.zeros((16,), jnp.float32))
    mxv[:] = mx

    for c in range(4):
        pltpu.sync_copy(sL.at[c], Ss.at[pl.ds(c * _NPAD + base, _PP)])
        pltpu.sync_copy(bL.at[c],
                        Sbox.at[pl.ds((c * (_NPAD // 16) + wid * _NBLK) * 64,
                                      _NBLK * 64)])
        pltpu.sync_copy(l1L.at[pl.ds(c * _NBLK, _NBLK)],
                        SL1.at[pl.ds(c * (_NPAD // 16) + wid * _NBLK, _NBLK)])
    pltpu.sync_copy(mxv, Smax.at[pl.ds(wid * 16, 16)])

    plsc.subcore_barrier()

    @pl.when(wid == 0)
    def _scan():
        pltpu.sync_copy(Ss, sc_v)
        pltpu.sync_copy(SL1, L1)
        pltpu.sync_copy(Smax, smax_v)
        m16 = smax_v[pl.ds(0, 16)]
        for i in range(1, _NW):
            m16 = jnp.maximum(m16, smax_v[pl.ds(i * 16, 16)])
        off_base = jnp.max(m16) + 1.0

        zf = jnp.zeros((16,), jnp.float32)
        zi = jnp.zeros((16,), jnp.int32)
        for j in range(8):
            sl = pl.ds(j * 16, 16)
            ox1[sl] = zf
            oy1[sl] = zf
            ox2[sl] = zf
            oy2[sl] = zf
            osc[sl] = zf
            olab[sl] = zi

        def bl2(chunk, _):
            idxb = chunk * 256 + lane * 16
            m = plsc.load_gather(L1, [idxb])
            for i in range(1, 16):
                m = jnp.maximum(m, plsc.load_gather(L1, [idxb + i]))
            L2r[pl.ds(chunk * 16, 16)] = m
            return 0

        lax.fori_loop(0, _L2 // 16, bl2, 0)

        for j in range(6):
            L2r[pl.ds(_L2 + j * 16, 16)] = jnp.full((16,), -2.0, jnp.float32)

        idxb = lane * 16
        m = plsc.load_gather(L2r, [idxb])
        for i in range(1, 16):
            m = jnp.maximum(m, plsc.load_gather(L2r, [idxb + i]))
        L3r[:] = m

        big = jnp.int32(9999)

        def cond(st):
            nk, stop = st
            return (nk < _DETS) & (stop == 0)

        def body(st):
            nk, stop = st
            v3 = L3r[:]
            g3 = jnp.max(v3)
            valid = g3 > 0.0
            l3 = plsc.all_reduce_ffs(v3 == g3)[0]
            v2 = L2r[pl.ds(l3 * 16, 16)]
            l2 = plsc.all_reduce_ffs(v2 == g3)[0]
            e2 = l3 * 16 + l2
            v1 = L1[pl.ds(e2 * 16, 16)]
            l1 = plsc.all_reduce_ffs(v1 == g3)[0]
            e1 = e2 * 16 + l1
            cp = pltpu.async_copy(Sbox.at[pl.ds(e1 * 64, 64)], bbuf, sem)
            v0 = sc_v[pl.ds(e1 * 16, 16)]
            l0 = plsc.all_reduce_ffs(v0 == g3)[0]
            k = e1 * 16 + l0

            @pl.when(valid)
            def _consume():
                v0n = jnp.where(lane == l0, -1.0, v0)
                _lane0_store(sc_v, k, jnp.float32(-1.0), lane)
                _lane0_store(L1, e1, jnp.max(v0n), lane)
                _lane0_store(L2r, e2, jnp.max(L1[pl.ds(e2 * 16, 16)]), lane)
                _lane0_store(L3r, l3, jnp.max(L2r[pl.ds(l3 * 16, 16)]), lane)

            cp.wait()

            def spl(q):
                return plsc.load_gather(bbuf, [jnp.full((16,), q, jnp.int32)])

            bo1 = spl(l0)
            bo2 = spl(l0 + 16)
            bo3 = spl(l0 + 32)
            bo4 = spl(l0 + 48)
            lab = k // _NPAD + 1
            off = lab.astype(jnp.float32) * off_base
            bx1 = bo1 + off
            by1 = bo2 + off
            bx2 = bo3 + off
            by2 = bo4 + off
            areac = (bx2 - bx1) * (by2 - by1)

            nblk = (nk + 15) // 16

            def iou_blk(j, hit):
                sl = pl.ds(j * 16, 16)
                kx1 = kx1r[sl]
                ky1 = ky1r[sl]
                kx2 = kx2r[sl]
                ky2 = ky2r[sl]
                kar = karr[sl]
                ltx = jnp.maximum(kx1, bx1)
                lty = jnp.maximum(ky1, by1)
                rbx = jnp.minimum(kx2, bx2)
                rby = jnp.minimum(ky2, by2)
                inter = (jnp.maximum(rbx - ltx, 0.0)
                         * jnp.maximum(rby - lty, 0.0))
                iou = inter / (kar + areac - inter + 1e-9)
                ok = (j * 16 + lane) < nk
                return hit | ((iou > _NMS_THRESH) & ok).astype(jnp.int32)

            hit = lax.fori_loop(0, nblk, iou_blk, jnp.zeros((16,), jnp.int32))
            rej = plsc.all_reduce_population_count(hit > 0)[0] > 0
            keep_it = valid & (~rej)

            @pl.when(keep_it)
            def _append():
                _lane0_store(kx1r, nk, bx1, lane)
                _lane0_store(ky1r, nk, by1, lane)
                _lane0_store(kx2r, nk, bx2, lane)
                _lane0_store(ky2r, nk, by2, lane)
                _lane0_store(karr, nk, areac, lane)
                _lane0_store(ox1, nk, bo1, lane)
                _lane0_store(oy1, nk, bo2, lane)
                _lane0_store(ox2, nk, bo3, lane)
                _lane0_store(oy2, nk, bo4, lane)
                _lane0_store(osc, nk, g3, lane)
                _lane0_store(olab, nk, lab, lane)

            nk2 = nk + keep_it.astype(jnp.int32)
            stop2 = jnp.where(valid, 0, 1).astype(jnp.int32)
            return nk2, stop2

        lax.while_loop(cond, body, (jnp.int32(0), jnp.int32(0)))

        pltpu.sync_copy(ox1, ox1_h)
        pltpu.sync_copy(oy1, oy1_h)
        pltpu.sync_copy(ox2, ox2_h)
        pltpu.sync_copy(oy2, oy2_h)
        pltpu.sync_copy(osc, osc_h)
        pltpu.sync_copy(olab, olab_h)


@jax.jit
def kernel(class_logits, box_regression, proposals):
    padn = _NPAD - _N
    clT = jnp.pad(class_logits, ((0, padn), (0, 0))).T
    brT = jnp.pad(box_regression, ((0, padn), (0, 0))).T
    prT = jnp.pad(proposals, ((0, padn), (0, 0))).T

    mesh = plsc.VectorSubcoreMesh(core_axis_name="c", subcore_axis_name="s",
                                  num_cores=1, num_subcores=_NW)
    f32 = jnp.float32
    run = pl.kernel(
        _sc_body,
        out_type=(
            jax.ShapeDtypeStruct((128,), f32),
            jax.ShapeDtypeStruct((128,), f32),
            jax.ShapeDtypeStruct((128,), f32),
            jax.ShapeDtypeStruct((128,), f32),
            jax.ShapeDtypeStruct((128,), f32),
            jax.ShapeDtypeStruct((128,), jnp.int32),
        ),
        mesh=mesh,
        compiler_params=pltpu.CompilerParams(needs_layout_passes=False),
        scratch_types=[
            pltpu.VMEM((5, _PP), f32),
            pltpu.VMEM((20, _PP), f32),
            pltpu.VMEM((4, _PP), f32),
            pltpu.VMEM((4, _PP), f32),
            pltpu.VMEM((4, _NBLK * 64), f32),
            pltpu.VMEM((16,), f32),
            pltpu.VMEM((4 * _NBLK,), f32),
            pltpu.VMEM_SHARED((_CAND,), f32),
            pltpu.VMEM_SHARED((_LEAF,), f32),
            pltpu.VMEM_SHARED((_LEAF * 64,), f32),
            pltpu.VMEM_SHARED((_NW * 16,), f32),
            pltpu.VMEM((_CAND,), f32),
            pltpu.VMEM((_LEAF,), f32),
            pltpu.VMEM((256,), f32),
            pltpu.VMEM((16,), f32),
            pltpu.VMEM((_NW * 16,), f32),
            pltpu.VMEM((64,), f32),
            pltpu.SemaphoreType.DMA,
            pltpu.VMEM((112,), f32),
            pltpu.VMEM((112,), f32),
            pltpu.VMEM((112,), f32),
            pltpu.VMEM((112,), f32),
            pltpu.VMEM((112,), f32),
            pltpu.VMEM((128,), f32),
            pltpu.VMEM((128,), f32),
            pltpu.VMEM((128,), f32),
            pltpu.VMEM((128,), f32),
            pltpu.VMEM((128,), f32),
            pltpu.VMEM((128,), jnp.int32),
        ],
    )
    x1, y1, x2, y2, sc, lab = run(clT, brT, prT)
    out_boxes = jnp.stack([x1, y1, x2, y2], axis=1)[:_DETS]
    return out_boxes, sc[:_DETS], lab[:_DETS]


# speedup vs baseline: 1.6030x; 1.0313x over previous
import functools
import jax
import jax.numpy as jnp
from jax import lax
from jax.experimental import pallas as pl
from jax.experimental.pallas import tpu as pltpu
from jax.experimental.pallas import tpu_sc as plsc
import numpy as np

_N = 10000
_NPAD = 10240
_NW = 16
_PP = _NPAD // _NW
_NBLK = _PP // 16
_CAND = 4 * _NPAD
_LEAF = _CAND // 16
_L2 = _LEAF // 16
_SCORE_THRESH = 0.05
_NMS_THRESH = 0.5
_DETS = 100
_IMG = 512.0
_CLIP = float(np.log(1000.0 / 16.0))


def _lane0_store(ref, i, val, lane):
    idx = jnp.full((16,), i, jnp.int32)
    if getattr(val, "shape", ()) != (16,):
        val = jnp.full((16,), val, ref.dtype)
    plsc.store_scatter(ref, [idx], val, mask=lane == 0)


def _sc_body(cl_h, br_h, pr_h,
             ox1_h, oy1_h, ox2_h, oy2_h, osc_h, olab_h,
             cl_v, br_v, pr_v, sL, bL, mxv, l1L,
             Ss, SL1, Sbox, Smax,
             sc_v, L1, L2r, L3r, smax_v, bbuf, sem,
             kx1r, ky1r, kx2r, ky2r, karr,
             ox1, oy1, ox2, oy2, osc, olab):
    wid = lax.axis_index("s")
    lane = lax.broadcasted_iota(jnp.int32, (16,), 0)

    base = wid * _PP
    c1 = pltpu.async_copy(cl_h.at[:, pl.ds(base, _PP)], cl_v, sem)
    c2 = pltpu.async_copy(br_h.at[:, pl.ds(base, _PP)], br_v, sem)
    c3 = pltpu.async_copy(pr_h.at[:, pl.ds(base, _PP)], pr_v, sem)
    c1.wait()
    c2.wait()
    c3.wait()

    def pa(g, mx):
        sl = pl.ds(g * 16, 16)
        p0 = pr_v[0, sl]
        p1 = pr_v[1, sl]
        p2 = pr_v[2, sl]
        p3 = pr_v[3, sl]
        w = p2 - p0
        h = p3 - p1
        cx = p0 + 0.5 * w
        cy = p1 + 0.5 * h
        l0 = cl_v[0, sl]
        l1 = cl_v[1, sl]
        l2 = cl_v[2, sl]
        l3 = cl_v[3, sl]
        l4 = cl_v[4, sl]
        lm = jnp.maximum(jnp.maximum(jnp.maximum(l0, l1), jnp.maximum(l2, l3)), l4)
        e0 = jnp.exp(l0 - lm)
        e1 = jnp.exp(l1 - lm)
        e2 = jnp.exp(l2 - lm)
        e3 = jnp.exp(l3 - lm)
        e4 = jnp.exp(l4 - lm)
        den = e0 + e1 + e2 + e3 + e4
        real = (base + g * 16 + lane) < _N
        es = [e1, e2, e3, e4]
        for c in range(1, 5):
            dx = br_v[4 * c + 0, sl] / 10.0
            dy = br_v[4 * c + 1, sl] / 10.0
            dw = jnp.minimum(br_v[4 * c + 2, sl] / 5.0, _CLIP)
            dh = jnp.minimum(br_v[4 * c + 3, sl] / 5.0, _CLIP)
            px = dx * w + cx
            py = dy * h + cy
            pw = jnp.exp(dw) * w
            ph = jnp.exp(dh) * h
            x1 = jnp.clip(px - 0.5 * pw, 0.0, _IMG)
            x2 = jnp.clip(px + 0.5 * pw, 0.0, _IMG)
            y1 = jnp.clip(py - 0.5 * ph, 0.0, _IMG)
            y2 = jnp.clip(py + 0.5 * ph, 0.0, _IMG)
            scr = es[c - 1] / den
            keep = ((scr > _SCORE_THRESH) & ((x2 - x1) >= 0.01)
                    & ((y2 - y1) >= 0.01) & real)
            ms = jnp.where(keep, scr, -1.0)
            _lane0_store(l1L, (c - 1) * _NBLK + g, jnp.max(ms), lane)
            sL[c - 1, sl] = ms
            bL[c - 1, pl.ds(g * 64, 16)] = x1
            bL[c - 1, pl.ds(g * 64 + 16, 16)] = y1
            bL[c - 1, pl.ds(g * 64 + 32, 16)] = x2
            bL[c - 1, pl.ds(g * 64 + 48, 16)] = y2
            mx = jnp.maximum(mx, jnp.maximum(jnp.maximum(x1, x2),
                                             jnp.maximum(y1, y2)))
        return mx

    mx = lax.fori_loop(0, _NBLK, pa, jnp.zeros((16,), jnp.float32))
    mxv[:] = mx

    cps = []
    for c in range(4):
        cps.append(pltpu.async_copy(
            sL.at[c], Ss.at[pl.ds(c * _NPAD + base, _PP)], sem))
        cps.append(pltpu.async_copy(
            bL.at[c],
            Sbox.at[pl.ds((c * (_NPAD // 16) + wid * _NBLK) * 64,
                          _NBLK * 64)], sem))
        cps.append(pltpu.async_copy(
            l1L.at[pl.ds(c * _NBLK, _NBLK)],
            SL1.at[pl.ds(c * (_NPAD // 16) + wid * _NBLK, _NBLK)], sem))
    cps.append(pltpu.async_copy(mxv, Smax.at[pl.ds(wid * 16, 16)], sem))
    for cp in cps:
        cp.wait()

    plsc.subcore_barrier()

    @pl.when(wid == 0)
    def _scan():
        pltpu.sync_copy(Ss, sc_v)
        pltpu.sync_copy(SL1, L1)
        pltpu.sync_copy(Smax, smax_v)
        m16 = smax_v[pl.ds(0, 16)]
        for i in range(1, _NW):
            m16 = jnp.maximum(m16, smax_v[pl.ds(i * 16, 16)])
        off_base = jnp.max(m16) + 1.0

        zf = jnp.zeros((16,), jnp.float32)
        zi = jnp.zeros((16,), jnp.int32)
        for j in range(8):
            sl = pl.ds(j * 16, 16)
            ox1[sl] = zf
            oy1[sl] = zf
            ox2[sl] = zf
            oy2[sl] = zf
            osc[sl] = zf
            olab[sl] = zi

        def bl2(chunk, _):
            idxb = chunk * 256 + lane * 16
            m = plsc.load_gather(L1, [idxb])
            for i in range(1, 16):
                m = jnp.maximum(m, plsc.load_gather(L1, [idxb + i]))
            L2r[pl.ds(chunk * 16, 16)] = m
            return 0

        lax.fori_loop(0, _L2 // 16, bl2, 0)

        for j in range(6):
            L2r[pl.ds(_L2 + j * 16, 16)] = jnp.full((16,), -2.0, jnp.float32)

        idxb = lane * 16
        m = plsc.load_gather(L2r, [idxb])
        for i in range(1, 16):
            m = jnp.maximum(m, plsc.load_gather(L2r, [idxb + i]))
        L3r[:] = m

        big = jnp.int32(9999)

        def cond(st):
            nk, stop = st
            return (nk < _DETS) & (stop == 0)

        def body(st):
            nk, stop = st
            v3 = L3r[:]
            g3 = jnp.max(v3)
            valid = g3 > 0.0
            l3 = plsc.all_reduce_ffs(v3 == g3)[0]
            v2 = L2r[pl.ds(l3 * 16, 16)]
            l2 = plsc.all_reduce_ffs(v2 == g3)[0]
            e2 = l3 * 16 + l2
            v1 = L1[pl.ds(e2 * 16, 16)]
            l1 = plsc.all_reduce_ffs(v1 == g3)[0]
            e1 = e2 * 16 + l1
            cp = pltpu.async_copy(Sbox.at[pl.ds(e1 * 64, 64)], bbuf, sem)
            v0 = sc_v[pl.ds(e1 * 16, 16)]
            l0 = plsc.all_reduce_ffs(v0 == g3)[0]
            k = e1 * 16 + l0

            @pl.when(valid)
            def _consume():
                v0n = jnp.where(lane == l0, -1.0, v0)
                _lane0_store(sc_v, k, jnp.float32(-1.0), lane)
                _lane0_store(L1, e1, jnp.max(v0n), lane)
                _lane0_store(L2r, e2, jnp.max(L1[pl.ds(e2 * 16, 16)]), lane)
                _lane0_store(L3r, l3, jnp.max(L2r[pl.ds(l3 * 16, 16)]), lane)

            cp.wait()

            def spl(q):
                return plsc.load_gather(bbuf, [jnp.full((16,), q, jnp.int32)])

            bo1 = spl(l0)
            bo2 = spl(l0 + 16)
            bo3 = spl(l0 + 32)
            bo4 = spl(l0 + 48)
            lab = k // _NPAD + 1
            off = lab.astype(jnp.float32) * off_base
            bx1 = bo1 + off
            by1 = bo2 + off
            bx2 = bo3 + off
            by2 = bo4 + off
            areac = (bx2 - bx1) * (by2 - by1)

            nblk = (nk + 15) // 16

            def iou_blk(j, hit):
                sl = pl.ds(j * 16, 16)
                kx1 = kx1r[sl]
                ky1 = ky1r[sl]
                kx2 = kx2r[sl]
                ky2 = ky2r[sl]
                kar = karr[sl]
                ltx = jnp.maximum(kx1, bx1)
                lty = jnp.maximum(ky1, by1)
                rbx = jnp.minimum(kx2, bx2)
                rby = jnp.minimum(ky2, by2)
                inter = (jnp.maximum(rbx - ltx, 0.0)
                         * jnp.maximum(rby - lty, 0.0))
                iou = inter / (kar + areac - inter + 1e-9)
                ok = (j * 16 + lane) < nk
                return hit | ((iou > _NMS_THRESH) & ok).astype(jnp.int32)

            hit = lax.fori_loop(0, nblk, iou_blk, jnp.zeros((16,), jnp.int32))
            rej = plsc.all_reduce_population_count(hit > 0)[0] > 0
            keep_it = valid & (~rej)

            @pl.when(keep_it)
            def _append():
                _lane0_store(kx1r, nk, bx1, lane)
                _lane0_store(ky1r, nk, by1, lane)
                _lane0_store(kx2r, nk, bx2, lane)
                _lane0_store(ky2r, nk, by2, lane)
                _lane0_store(karr, nk, areac, lane)
                _lane0_store(ox1, nk, bo1, lane)
                _lane0_store(oy1, nk, bo2, lane)
                _lane0_store(ox2, nk, bo3, lane)
                _lane0_store(oy2, nk, bo4, lane)
                _lane0_store(osc, nk, g3, lane)
                _lane0_store(olab, nk, lab, lane)

            nk2 = nk + keep_it.astype(jnp.int32)
            stop2 = jnp.where(valid, 0, 1).astype(jnp.int32)
            return nk2, stop2

        lax.while_loop(cond, body, (jnp.int32(0), jnp.int32(0)))

        pltpu.sync_copy(ox1, ox1_h)
        pltpu.sync_copy(oy1, oy1_h)
        pltpu.sync_copy(ox2, ox2_h)
        pltpu.sync_copy(oy2, oy2_h)
        pltpu.sync_copy(osc, osc_h)
        pltpu.sync_copy(olab, olab_h)


@jax.jit
def kernel(class_logits, box_regression, proposals):
    padn = _NPAD - _N
    clT = jnp.pad(class_logits, ((0, padn), (0, 0))).T
    brT = jnp.pad(box_regression, ((0, padn), (0, 0))).T
    prT = jnp.pad(proposals, ((0, padn), (0, 0))).T

    mesh = plsc.VectorSubcoreMesh(core_axis_name="c", subcore_axis_name="s",
                                  num_cores=1, num_subcores=_NW)
    f32 = jnp.float32
    run = pl.kernel(
        _sc_body,
        out_type=(
            jax.ShapeDtypeStruct((128,), f32),
            jax.ShapeDtypeStruct((128,), f32),
            jax.ShapeDtypeStruct((128,), f32),
            jax.ShapeDtypeStruct((128,), f32),
            jax.ShapeDtypeStruct((128,), f32),
            jax.ShapeDtypeStruct((128,), jnp.int32),
        ),
        mesh=mesh,
        compiler_params=pltpu.CompilerParams(needs_layout_passes=False),
        scratch_types=[
            pltpu.VMEM((5, _PP), f32),
            pltpu.VMEM((20, _PP), f32),
            pltpu.VMEM((4, _PP), f32),
            pltpu.VMEM((4, _PP), f32),
            pltpu.VMEM((4, _NBLK * 64), f32),
            pltpu.VMEM((16,), f32),
            pltpu.VMEM((4 * _NBLK,), f32),
            pltpu.VMEM_SHARED((_CAND,), f32),
            pltpu.VMEM_SHARED((_LEAF,), f32),
            pltpu.VMEM_SHARED((_LEAF * 64,), f32),
            pltpu.VMEM_SHARED((_NW * 16,), f32),
            pltpu.VMEM((_CAND,), f32),
            pltpu.VMEM((_LEAF,), f32),
            pltpu.VMEM((256,), f32),
            pltpu.VMEM((16,), f32),
            pltpu.VMEM((_NW * 16,), f32),
            pltpu.VMEM((64,), f32),
            pltpu.SemaphoreType.DMA,
            pltpu.VMEM((112,), f32),
            pltpu.VMEM((112,), f32),
            pltpu.VMEM((112,), f32),
            pltpu.VMEM((112,), f32),
            pltpu.VMEM((112,), f32),
            pltpu.VMEM((128,), f32),
            pltpu.VMEM((128,), f32),
            pltpu.VMEM((128,), f32),
            pltpu.VMEM((128,), f32),
            pltpu.VMEM((128,), f32),
            pltpu.VMEM((128,), jnp.int32),
        ],
    )
    x1, y1, x2, y2, sc, lab = run(clT, brT, prT)
    out_boxes = jnp.stack([x1, y1, x2, y2], axis=1)[:_DETS]
    return out_boxes, sc[:_DETS], lab[:_DETS]


# speedup vs baseline: 1.6041x; 1.0007x over previous
import jax
import jax.numpy as jnp
from jax import lax
from jax.experimental import pallas as pl
from jax.experimental.pallas import tpu as pltpu
from jax.experimental.pallas import tpu_sc as plsc
import numpy as np

_N = 10000
_NPAD = 10240
_NW = 16
_PP = _NPAD // _NW
_NBLK = _PP // 16
_CAND = 4 * _NPAD
_LEAF = _CAND // 16
_L2 = _LEAF // 16
_SCORE_THRESH = 0.05
_NMS_THRESH = 0.5
_DETS = 100
_IMG = 512.0
_CLIP = float(np.log(1000.0 / 16.0))


def _lane0_store(ref, i, val, lane):
    idx = jnp.full((16,), i, jnp.int32)
    if getattr(val, "shape", ()) != (16,):
        val = jnp.full((16,), val, ref.dtype)
    plsc.store_scatter(ref, [idx], val, mask=lane == 0)


def _sc_body(cl_h, br_h, pr_h,
             ox1_h, oy1_h, ox2_h, oy2_h, osc_h, olab_h,
             cl_v, br_v, pr_v, sL, bL, mxv, l1L,
             Ss, SL1, Sbox, Smax,
             sc_v, L1, L2r, L3r, smax_v, bbuf, sem,
             kx1r, ky1r, kx2r, ky2r, karr,
             ox1, oy1, ox2, oy2, osc, olab):
    wid = lax.axis_index("s")
    lane = lax.broadcasted_iota(jnp.int32, (16,), 0)

    base = wid * _PP
    c1 = pltpu.async_copy(cl_h.at[:, pl.ds(base, _PP)], cl_v, sem)
    c2 = pltpu.async_copy(br_h.at[:, pl.ds(base, _PP)], br_v, sem)
    c3 = pltpu.async_copy(pr_h.at[:, pl.ds(base, _PP)], pr_v, sem)
    c1.wait()
    c2.wait()
    c3.wait()

    def pa(g, mx):
        sl = pl.ds(g * 16, 16)
        p0 = pr_v[0, sl]
        p1 = pr_v[1, sl]
        p2 = pr_v[2, sl]
        p3 = pr_v[3, sl]
        w = p2 - p0
        h = p3 - p1
        cx = p0 + 0.5 * w
        cy = p1 + 0.5 * h
        l0 = cl_v[0, sl]
        l1 = cl_v[1, sl]
        l2 = cl_v[2, sl]
        l3 = cl_v[3, sl]
        l4 = cl_v[4, sl]
        lm = jnp.maximum(jnp.maximum(jnp.maximum(l0, l1), jnp.maximum(l2, l3)), l4)
        e0 = jnp.exp(l0 - lm)
        e1 = jnp.exp(l1 - lm)
        e2 = jnp.exp(l2 - lm)
        e3 = jnp.exp(l3 - lm)
        e4 = jnp.exp(l4 - lm)
        den = e0 + e1 + e2 + e3 + e4
        real = (base + g * 16 + lane) < _N
        es = [e1, e2, e3, e4]
        for c in range(1, 5):
            dx = br_v[4 * c + 0, sl] / 10.0
            dy = br_v[4 * c + 1, sl] / 10.0
            dw = jnp.minimum(br_v[4 * c + 2, sl] / 5.0, _CLIP)
            dh = jnp.minimum(br_v[4 * c + 3, sl] / 5.0, _CLIP)
            px = dx * w + cx
            py = dy * h + cy
            pw = jnp.exp(dw) * w
            ph = jnp.exp(dh) * h
            x1 = jnp.clip(px - 0.5 * pw, 0.0, _IMG)
            x2 = jnp.clip(px + 0.5 * pw, 0.0, _IMG)
            y1 = jnp.clip(py - 0.5 * ph, 0.0, _IMG)
            y2 = jnp.clip(py + 0.5 * ph, 0.0, _IMG)
            scr = es[c - 1] / den
            keep = ((scr > _SCORE_THRESH) & ((x2 - x1) >= 0.01)
                    & ((y2 - y1) >= 0.01) & real)
            ms = jnp.where(keep, scr, -1.0)
            _lane0_store(l1L, (c - 1) * _NBLK + g, jnp.max(ms), lane)
            sL[c - 1, sl] = ms
            bL[c - 1, pl.ds(g * 64, 16)] = x1
            bL[c - 1, pl.ds(g * 64 + 16, 16)] = y1
            bL[c - 1, pl.ds(g * 64 + 32, 16)] = x2
            bL[c - 1, pl.ds(g * 64 + 48, 16)] = y2
            mx = jnp.maximum(mx, jnp.maximum(jnp.maximum(x1, x2),
                                             jnp.maximum(y1, y2)))
        return mx

    mx = lax.fori_loop(0, _NBLK, pa, jnp.zeros((16,), jnp.float32))
    mxv[:] = mx

    cps = []
    for c in range(4):
        cps.append(pltpu.async_copy(
            sL.at[c], Ss.at[pl.ds(c * _NPAD + base, _PP)], sem))
        cps.append(pltpu.async_copy(
            bL.at[c],
            Sbox.at[pl.ds((c * (_NPAD // 16) + wid * _NBLK) * 64,
                          _NBLK * 64)], sem))
        cps.append(pltpu.async_copy(
            l1L.at[pl.ds(c * _NBLK, _NBLK)],
            SL1.at[pl.ds(c * (_NPAD // 16) + wid * _NBLK, _NBLK)], sem))
    cps.append(pltpu.async_copy(mxv, Smax.at[pl.ds(wid * 16, 16)], sem))
    for cp in cps:
        cp.wait()

    plsc.subcore_barrier()

    @pl.when(wid == 0)
    def _scan():
        pltpu.sync_copy(Ss, sc_v)
        pltpu.sync_copy(SL1, L1)
        pltpu.sync_copy(Smax, smax_v)
        m16 = smax_v[pl.ds(0, 16)]
        for i in range(1, _NW):
            m16 = jnp.maximum(m16, smax_v[pl.ds(i * 16, 16)])
        off_base = jnp.max(m16) + 1.0

        zf = jnp.zeros((16,), jnp.float32)
        zi = jnp.zeros((16,), jnp.int32)
        for j in range(8):
            sl = pl.ds(j * 16, 16)
            ox1[sl] = zf
            oy1[sl] = zf
            ox2[sl] = zf
            oy2[sl] = zf
            osc[sl] = zf
            olab[sl] = zi

        def bl2(chunk, _):
            idxb = chunk * 256 + lane * 16
            m = plsc.load_gather(L1, [idxb])
            for i in range(1, 16):
                m = jnp.maximum(m, plsc.load_gather(L1, [idxb + i]))
            L2r[pl.ds(chunk * 16, 16)] = m
            return 0

        lax.fori_loop(0, _L2 // 16, bl2, 0)

        for j in range(6):
            L2r[pl.ds(_L2 + j * 16, 16)] = jnp.full((16,), -2.0, jnp.float32)

        idxb = lane * 16
        m = plsc.load_gather(L2r, [idxb])
        for i in range(1, 16):
            m = jnp.maximum(m, plsc.load_gather(L2r, [idxb + i]))
        L3r[:] = m

        def cond(st):
            nk, stop = st
            return (nk < _DETS) & (stop == 0)

        def body(st):
            nk, stop = st
            v3 = L3r[:]
            g3 = jnp.max(v3)
            valid = g3 > 0.0
            l3 = plsc.all_reduce_ffs(v3 == g3)[0]
            v2 = L2r[pl.ds(l3 * 16, 16)]
            l2 = plsc.all_reduce_ffs(v2 == g3)[0]
            e2 = l3 * 16 + l2
            v1 = L1[pl.ds(e2 * 16, 16)]
            l1 = plsc.all_reduce_ffs(v1 == g3)[0]
            e1 = e2 * 16 + l1
            cp = pltpu.async_copy(Sbox.at[pl.ds(e1 * 64, 64)], bbuf, sem)
            v0 = sc_v[pl.ds(e1 * 16, 16)]
            l0 = plsc.all_reduce_ffs(v0 == g3)[0]
            k = e1 * 16 + l0

            @pl.when(valid)
            def _consume():
                v0n = jnp.where(lane == l0, -1.0, v0)
                _lane0_store(sc_v, k, jnp.float32(-1.0), lane)
                _lane0_store(L1, e1, jnp.max(v0n), lane)
                _lane0_store(L2r, e2, jnp.max(L1[pl.ds(e2 * 16, 16)]), lane)
                _lane0_store(L3r, l3, jnp.max(L2r[pl.ds(l3 * 16, 16)]), lane)

            cp.wait()

            def spl(q):
                return plsc.load_gather(bbuf, [jnp.full((16,), q, jnp.int32)])

            bo1 = spl(l0)
            bo2 = spl(l0 + 16)
            bo3 = spl(l0 + 32)
            bo4 = spl(l0 + 48)
            lab = k // _NPAD + 1
            off = lab.astype(jnp.float32) * off_base
            bx1 = bo1 + off
            by1 = bo2 + off
            bx2 = bo3 + off
            by2 = bo4 + off
            areac = (bx2 - bx1) * (by2 - by1)

            nblk = (nk + 15) // 16

            def iou_blk(j, hit):
                sl = pl.ds(j * 16, 16)
                kx1 = kx1r[sl]
                ky1 = ky1r[sl]
                kx2 = kx2r[sl]
                ky2 = ky2r[sl]
                kar = karr[sl]
                ltx = jnp.maximum(kx1, bx1)
                lty = jnp.maximum(ky1, by1)
                rbx = jnp.minimum(kx2, bx2)
                rby = jnp.minimum(ky2, by2)
                inter = (jnp.maximum(rbx - ltx, 0.0)
                         * jnp.maximum(rby - lty, 0.0))
                iou = inter / (kar + areac - inter + 1e-9)
                ok = (j * 16 + lane) < nk
                return hit | ((iou > _NMS_THRESH) & ok).astype(jnp.int32)

            hit = lax.fori_loop(0, nblk, iou_blk, jnp.zeros((16,), jnp.int32))
            rej = plsc.all_reduce_population_count(hit > 0)[0] > 0
            keep_it = valid & (~rej)

            @pl.when(keep_it)
            def _append():
                _lane0_store(kx1r, nk, bx1, lane)
                _lane0_store(ky1r, nk, by1, lane)
                _lane0_store(kx2r, nk, bx2, lane)
                _lane0_store(ky2r, nk, by2, lane)
                _lane0_store(karr, nk, areac, lane)
                _lane0_store(ox1, nk, bo1, lane)
                _lane0_store(oy1, nk, bo2, lane)
                _lane0_store(ox2, nk, bo3, lane)
                _lane0_store(oy2, nk, bo4, lane)
                _lane0_store(osc, nk, g3, lane)
                _lane0_store(olab, nk, lab, lane)

            nk2 = nk + keep_it.astype(jnp.int32)
            stop2 = jnp.where(valid, 0, 1).astype(jnp.int32)
            return nk2, stop2

        lax.while_loop(cond, body, (jnp.int32(0), jnp.int32(0)))

        pltpu.sync_copy(ox1, ox1_h)
        pltpu.sync_copy(oy1, oy1_h)
        pltpu.sync_copy(ox2, ox2_h)
        pltpu.sync_copy(oy2, oy2_h)
        pltpu.sync_copy(osc, osc_h)
        pltpu.sync_copy(olab, olab_h)


@jax.jit
def kernel(class_logits, box_regression, proposals):
    padn = _NPAD - _N
    clT = jnp.pad(class_logits, ((0, padn), (0, 0))).T
    brT = jnp.pad(box_regression, ((0, padn), (0, 0))).T
    prT = jnp.pad(proposals, ((0, padn), (0, 0))).T

    mesh = plsc.VectorSubcoreMesh(core_axis_name="c", subcore_axis_name="s",
                                  num_cores=1, num_subcores=_NW)
    f32 = jnp.float32
    run = pl.kernel(
        _sc_body,
        out_type=(
            jax.ShapeDtypeStruct((128,), f32),
            jax.ShapeDtypeStruct((128,), f32),
            jax.ShapeDtypeStruct((128,), f32),
            jax.ShapeDtypeStruct((128,), f32),
            jax.ShapeDtypeStruct((128,), f32),
            jax.ShapeDtypeStruct((128,), jnp.int32),
        ),
        mesh=mesh,
        compiler_params=pltpu.CompilerParams(needs_layout_passes=False),
        scratch_types=[
            pltpu.VMEM((5, _PP), f32),
            pltpu.VMEM((20, _PP), f32),
            pltpu.VMEM((4, _PP), f32),
            pltpu.VMEM((4, _PP), f32),
            pltpu.VMEM((4, _NBLK * 64), f32),
            pltpu.VMEM((16,), f32),
            pltpu.VMEM((4 * _NBLK,), f32),
            pltpu.VMEM_SHARED((_CAND,), f32),
            pltpu.VMEM_SHARED((_LEAF,), f32),
            pltpu.VMEM_SHARED((_LEAF * 64,), f32),
            pltpu.VMEM_SHARED((_NW * 16,), f32),
            pltpu.VMEM((_CAND,), f32),
            pltpu.VMEM((_LEAF,), f32),
            pltpu.VMEM((256,), f32),
            pltpu.VMEM((16,), f32),
            pltpu.VMEM((_NW * 16,), f32),
            pltpu.VMEM((64,), f32),
            pltpu.SemaphoreType.DMA,
            pltpu.VMEM((112,), f32),
            pltpu.VMEM((112,), f32),
            pltpu.VMEM((112,), f32),
            pltpu.VMEM((112,), f32),
            pltpu.VMEM((112,), f32),
            pltpu.VMEM((128,), f32),
            pltpu.VMEM((128,), f32),
            pltpu.VMEM((128,), f32),
            pltpu.VMEM((128,), f32),
            pltpu.VMEM((128,), f32),
            pltpu.VMEM((128,), jnp.int32),
        ],
    )
    x1, y1, x2, y2, sc, lab = run(clT, brT, prT)
    out_boxes = jnp.stack([x1, y1, x2, y2], axis=1)[:_DETS]
    return out_boxes, sc[:_DETS], lab[:_DETS]
